# Initial kernel scaffold; baseline (speedup 1.0000x reference)
import jax
import jax.numpy as jnp
from jax.experimental import pallas as pl


def kernel(x, edge_index, W_l1, b_l1, W_r1, W_l2, b_l2, W_r2):
    raise NotImplementedError("write your pallas kernel here")


# speedup vs baseline: 10.2676x; 10.2676x over previous
import functools

import jax
import jax.numpy as jnp
from jax import lax
from jax.experimental import pallas as pl
from jax.experimental.pallas import tpu as pltpu
from jax.experimental.pallas import tpu_sc as plsc

_N = 10000
_E = 320000
_D = 128
_H = 16
_C = 300

_NC = 2
_NS = 16
_NW = _NC * _NS
_CHUNK = 128
_NCHUNK = 80
_EPT = _NCHUNK * _CHUNK
_EPAD = _NW * _EPT
_NPAD = _NW * 320
_RPT = _NPAD // _NS


def _seg_body(with_deg, xl_hbm, src_hbm, dst_hbm, zrows_hbm, zdeg_hbm,
              *refs):
    if with_deg:
        (s_out, deg_out, s_sh, deg_sh, src_v, dst_v, rows_v, ones_v,
         tmp_rows, tmp_deg, sem) = refs
    else:
        (s_out, s_sh, src_v, dst_v, rows_v, tmp_rows, sem) = refs
    c = lax.axis_index("c")
    s = lax.axis_index("s")
    wid = c * _NS + s

    pltpu.sync_copy(zrows_hbm, tmp_rows)
    pltpu.sync_copy(tmp_rows, s_sh.at[pl.ds(s * _RPT, _RPT)])
    if with_deg:
        pltpu.sync_copy(zdeg_hbm, tmp_deg)
        pltpu.sync_copy(tmp_deg, deg_sh.at[pl.ds(s * _RPT, _RPT)])
        for i in range(_CHUNK // 16):
            ones_v[pl.ds(i * 16, 16)] = jnp.ones((16,), jnp.float32)

    pltpu.sync_copy(src_hbm.at[pl.ds(wid * _NCHUNK, _NCHUNK)], src_v)
    pltpu.sync_copy(dst_hbm.at[pl.ds(wid * _NCHUNK, _NCHUNK)], dst_v)
    plsc.subcore_barrier()

    def body(j, carry):
        pltpu.async_copy(xl_hbm.at[src_v.at[j]], rows_v, sem).wait()
        pltpu.sync_copy(rows_v, s_sh.at[dst_v.at[j]], add=True)
        if with_deg:
            pltpu.sync_copy(ones_v, deg_sh.at[dst_v.at[j]], add=True)
        return carry

    lax.fori_loop(0, _NCHUNK, body, 0)
    plsc.subcore_barrier()

    pltpu.sync_copy(s_sh.at[pl.ds(s * _RPT, _RPT)], tmp_rows)
    pltpu.sync_copy(tmp_rows, s_out.at[c, pl.ds(s * _RPT, _RPT)])
    if with_deg:
        pltpu.sync_copy(deg_sh.at[pl.ds(s * _RPT, _RPT)], tmp_deg)
        pltpu.sync_copy(tmp_deg, deg_out.at[c, pl.ds(s * _RPT, _RPT)])


def _make_seg_kernel(with_deg):
    mesh = plsc.VectorSubcoreMesh(core_axis_name="c", subcore_axis_name="s")
    if with_deg:
        out_type = (jax.ShapeDtypeStruct((_NC, _NPAD, _H), jnp.float32),
                    jax.ShapeDtypeStruct((_NC, _NPAD), jnp.float32))
    else:
        out_type = jax.ShapeDtypeStruct((_NC, _NPAD, _H), jnp.float32)
    scratch = [pltpu.VMEM_SHARED((_NPAD, _H), jnp.float32)]
    if with_deg:
        scratch.append(pltpu.VMEM_SHARED((_NPAD,), jnp.float32))
    scratch += [
        pltpu.VMEM((_NCHUNK, _CHUNK), jnp.int32),
        pltpu.VMEM((_NCHUNK, _CHUNK), jnp.int32),
        pltpu.VMEM((_CHUNK, _H), jnp.float32),
    ]
    if with_deg:
        scratch.append(pltpu.VMEM((_CHUNK,), jnp.float32))
    scratch.append(pltpu.VMEM((_RPT, _H), jnp.float32))
    if with_deg:
        scratch.append(pltpu.VMEM((_RPT,), jnp.float32))
    scratch.append(pltpu.SemaphoreType.DMA)
    return pl.kernel(functools.partial(_seg_body, with_deg),
                     out_type=out_type, mesh=mesh, scratch_types=scratch,
                     compiler_params=pltpu.CompilerParams(
                         use_tc_tiling_on_sc=False))


_seg_with_deg = _make_seg_kernel(True)
_seg_no_deg = _make_seg_kernel(False)


def _proj_body(x_ref, wl_ref, wr_ref, xl_ref, xr_ref):
    x = x_ref[...]
    xl_ref[...] = jnp.dot(x, wl_ref[...], preferred_element_type=jnp.float32)
    xr_ref[...] = jnp.dot(x, wr_ref[...], preferred_element_type=jnp.float32)


def _layer1_body(s1_ref, deg_ref, xr_ref, b_ref, h_ref, dinv_ref):
    deg = deg_ref[0] + deg_ref[1]
    dinv = 1.0 / jnp.maximum(deg, 1.0)
    s1 = s1_ref[0] + s1_ref[1]
    h_ref[...] = jnp.maximum(s1 * dinv + b_ref[...] + xr_ref[...], 0.0)
    dinv_ref[...] = dinv


def _layer2_body(s2_ref, dinv_ref, h_ref, wl_ref, b_ref, wr_ref, o_ref):
    agg = (s2_ref[0] + s2_ref[1]) * dinv_ref[...]
    z = (jnp.dot(agg, wl_ref[...], preferred_element_type=jnp.float32)
         + b_ref[...]
         + jnp.dot(h_ref[...], wr_ref[...], preferred_element_type=jnp.float32))
    m = jnp.max(z, axis=1, keepdims=True)
    lse = jnp.log(jnp.sum(jnp.exp(z - m), axis=1, keepdims=True)) + m
    o_ref[...] = z - lse


_BN = 2000


def kernel(x, edge_index, W_l1, b_l1, W_r1, W_l2, b_l2, W_r2):
    f32 = jnp.float32
    src = edge_index[0]
    dst = edge_index[1]
    pad = _EPAD - _E
    src_p = jnp.concatenate([src, jnp.zeros((pad,), jnp.int32)])
    dst_p = jnp.concatenate([dst, jnp.full((pad,), _N, jnp.int32)])
    src2d = src_p.reshape(_EPAD // _CHUNK, _CHUNK)
    dst2d = dst_p.reshape(_EPAD // _CHUNK, _CHUNK)
    zrows = jnp.zeros((_RPT, _H), f32)
    zdeg = jnp.zeros((_RPT,), f32)

    xl, xr = pl.pallas_call(
        _proj_body,
        grid=(_N // _BN,),
        in_specs=[pl.BlockSpec((_BN, _D), lambda i: (i, 0)),
                  pl.BlockSpec((_D, _H), lambda i: (0, 0)),
                  pl.BlockSpec((_D, _H), lambda i: (0, 0))],
        out_specs=[pl.BlockSpec((_BN, _H), lambda i: (i, 0)),
                   pl.BlockSpec((_BN, _H), lambda i: (i, 0))],
        out_shape=[jax.ShapeDtypeStruct((_N, _H), f32),
                   jax.ShapeDtypeStruct((_N, _H), f32)],
    )(x, W_l1, W_r1)

    s1p, degp = _seg_with_deg(xl, src2d, dst2d, zrows, zdeg)
    s1p = s1p[:, :_N, :]
    degp = degp[:, :_N, None]

    h, dinv = pl.pallas_call(
        _layer1_body,
        grid=(_N // _BN,),
        in_specs=[pl.BlockSpec((2, _BN, _H), lambda i: (0, i, 0)),
                  pl.BlockSpec((2, _BN, 1), lambda i: (0, i, 0)),
                  pl.BlockSpec((_BN, _H), lambda i: (i, 0)),
                  pl.BlockSpec((1, _H), lambda i: (0, 0))],
        out_specs=[pl.BlockSpec((_BN, _H), lambda i: (i, 0)),
                   pl.BlockSpec((_BN, 1), lambda i: (i, 0))],
        out_shape=[jax.ShapeDtypeStruct((_N, _H), f32),
                   jax.ShapeDtypeStruct((_N, 1), f32)],
    )(s1p, degp, xr, b_l1.reshape(1, _H))

    s2p = _seg_no_deg(h, src2d, dst2d, zrows, zdeg)
    s2p = s2p[:, :_N, :]

    out = pl.pallas_call(
        _layer2_body,
        grid=(_N // _BN,),
        in_specs=[pl.BlockSpec((2, _BN, _H), lambda i: (0, i, 0)),
                  pl.BlockSpec((_BN, 1), lambda i: (i, 0)),
                  pl.BlockSpec((_BN, _H), lambda i: (i, 0)),
                  pl.BlockSpec((_H, _C), lambda i: (0, 0)),
                  pl.BlockSpec((1, _C), lambda i: (0, 0)),
                  pl.BlockSpec((_H, _C), lambda i: (0, 0))],
        out_specs=pl.BlockSpec((_BN, _C), lambda i: (i, 0)),
        out_shape=jax.ShapeDtypeStruct((_N, _C), f32),
    )(s2p, dinv, h, W_l2, b_l2.reshape(1, _C), W_r2)
    return out


# speedup vs baseline: 13.5054x; 1.3153x over previous
import functools

import jax
import jax.numpy as jnp
from jax import lax
from jax.experimental import pallas as pl
from jax.experimental.pallas import tpu as pltpu
from jax.experimental.pallas import tpu_sc as plsc

_N = 10000
_E = 320000
_D = 128
_H = 16
_C = 300

_NC = 2
_NS = 16
_NW = _NC * _NS
_CHUNK = 128
_NCHUNK = 80
_EPT = _NCHUNK * _CHUNK
_EPAD = _NW * _EPT
_NPAD = _NW * 320
_RPT = _NPAD // _NS


def _seg_body(with_deg, xl_hbm, src_hbm, dst_hbm, zrows_hbm, zdeg_hbm,
              *refs):
    if with_deg:
        (s_out, deg_out, s_sh, deg_sh, src_v, dst_v, rows_v, ones_v,
         tmp_rows, tmp_deg, gsem, ssem, dsem) = refs
    else:
        (s_out, s_sh, src_v, dst_v, rows_v, tmp_rows, gsem, ssem, dsem) = refs
    c = lax.axis_index("c")
    s = lax.axis_index("s")
    wid = c * _NS + s

    pltpu.sync_copy(zrows_hbm, tmp_rows)
    pltpu.sync_copy(tmp_rows, s_sh.at[pl.ds(s * _RPT, _RPT)])
    if with_deg:
        pltpu.sync_copy(zdeg_hbm, tmp_deg)
        pltpu.sync_copy(tmp_deg, deg_sh.at[pl.ds(s * _RPT, _RPT)])
        for i in range(_CHUNK // 16):
            ones_v[pl.ds(i * 16, 16)] = jnp.ones((16,), jnp.float32)

    pltpu.sync_copy(src_hbm.at[pl.ds(wid * _NCHUNK, _NCHUNK)], src_v)
    pltpu.sync_copy(dst_hbm.at[pl.ds(wid * _NCHUNK, _NCHUNK)], dst_v)
    plsc.subcore_barrier()

    pltpu.async_copy(xl_hbm.at[src_v.at[0]], rows_v.at[0], gsem.at[0])

    def body(j, carry):
        b = lax.rem(j, 2)
        nb = lax.rem(j + 1, 2)

        @pl.when(j + 1 < _NCHUNK)
        def _fire_next():
            @pl.when(j >= 1)
            def _drain_prev_scatter():
                pltpu.make_async_copy(
                    rows_v.at[nb], s_sh.at[dst_v.at[j]], ssem.at[nb]).wait()
            pltpu.async_copy(xl_hbm.at[src_v.at[j + 1]], rows_v.at[nb],
                             gsem.at[nb])

        pltpu.make_async_copy(xl_hbm.at[src_v.at[j]], rows_v.at[b],
                              gsem.at[b]).wait()
        pltpu.async_copy(rows_v.at[b], s_sh.at[dst_v.at[j]], ssem.at[b],
                         add=True)
        if with_deg:
            @pl.when(j >= 1)
            def _drain_prev_deg():
                pltpu.make_async_copy(ones_v, deg_sh.at[dst_v.at[j]],
                                      dsem).wait()
            pltpu.async_copy(ones_v, deg_sh.at[dst_v.at[j]], dsem, add=True)
        return carry

    lax.fori_loop(0, _NCHUNK, body, 0)
    for b in range(2):
        pltpu.make_async_copy(rows_v.at[b], s_sh.at[dst_v.at[0]],
                              ssem.at[b]).wait()
    if with_deg:
        pltpu.make_async_copy(ones_v, deg_sh.at[dst_v.at[0]], dsem).wait()
    plsc.subcore_barrier()

    pltpu.sync_copy(s_sh.at[pl.ds(s * _RPT, _RPT)], tmp_rows)
    pltpu.sync_copy(tmp_rows, s_out.at[c, pl.ds(s * _RPT, _RPT)])
    if with_deg:
        pltpu.sync_copy(deg_sh.at[pl.ds(s * _RPT, _RPT)], tmp_deg)
        pltpu.sync_copy(tmp_deg, deg_out.at[c, pl.ds(s * _RPT, _RPT)])


def _make_seg_kernel(with_deg):
    mesh = plsc.VectorSubcoreMesh(core_axis_name="c", subcore_axis_name="s")
    if with_deg:
        out_type = (jax.ShapeDtypeStruct((_NC, _NPAD, _H), jnp.float32),
                    jax.ShapeDtypeStruct((_NC, _NPAD), jnp.float32))
    else:
        out_type = jax.ShapeDtypeStruct((_NC, _NPAD, _H), jnp.float32)
    scratch = [pltpu.VMEM_SHARED((_NPAD, _H), jnp.float32)]
    if with_deg:
        scratch.append(pltpu.VMEM_SHARED((_NPAD,), jnp.float32))
    scratch += [
        pltpu.VMEM((_NCHUNK, _CHUNK), jnp.int32),
        pltpu.VMEM((_NCHUNK, _CHUNK), jnp.int32),
        pltpu.VMEM((2, _CHUNK, _H), jnp.float32),
    ]
    if with_deg:
        scratch.append(pltpu.VMEM((_CHUNK,), jnp.float32))
    scratch.append(pltpu.VMEM((_RPT, _H), jnp.float32))
    if with_deg:
        scratch.append(pltpu.VMEM((_RPT,), jnp.float32))
    scratch += [pltpu.SemaphoreType.DMA((2,)),
                pltpu.SemaphoreType.DMA((2,)),
                pltpu.SemaphoreType.DMA]
    return pl.kernel(functools.partial(_seg_body, with_deg),
                     out_type=out_type, mesh=mesh, scratch_types=scratch,
                     compiler_params=pltpu.CompilerParams(
                         use_tc_tiling_on_sc=False))


_seg_with_deg = _make_seg_kernel(True)
_seg_no_deg = _make_seg_kernel(False)


def _proj_body(x_ref, wl_ref, wr_ref, xl_ref, xr_ref):
    x = x_ref[...]
    xl_ref[...] = jnp.dot(x, wl_ref[...], preferred_element_type=jnp.float32)
    xr_ref[...] = jnp.dot(x, wr_ref[...], preferred_element_type=jnp.float32)


def _layer1_body(s1_ref, deg_ref, xr_ref, b_ref, h_ref, dinv_ref):
    deg = deg_ref[0] + deg_ref[1]
    dinv = 1.0 / jnp.maximum(deg, 1.0)
    s1 = s1_ref[0] + s1_ref[1]
    h_ref[...] = jnp.maximum(s1 * dinv + b_ref[...] + xr_ref[...], 0.0)
    dinv_ref[...] = dinv


def _layer2_body(s2_ref, dinv_ref, h_ref, wl_ref, b_ref, wr_ref, o_ref):
    agg = (s2_ref[0] + s2_ref[1]) * dinv_ref[...]
    z = (jnp.dot(agg, wl_ref[...], preferred_element_type=jnp.float32)
         + b_ref[...]
         + jnp.dot(h_ref[...], wr_ref[...], preferred_element_type=jnp.float32))
    m = jnp.max(z, axis=1, keepdims=True)
    lse = jnp.log(jnp.sum(jnp.exp(z - m), axis=1, keepdims=True)) + m
    o_ref[...] = z - lse


_BN = 2000


def kernel(x, edge_index, W_l1, b_l1, W_r1, W_l2, b_l2, W_r2):
    f32 = jnp.float32
    src = edge_index[0]
    dst = edge_index[1]
    pad = _EPAD - _E
    src_p = jnp.concatenate([src, jnp.zeros((pad,), jnp.int32)])
    dst_p = jnp.concatenate([dst, jnp.full((pad,), _N, jnp.int32)])
    src2d = src_p.reshape(_EPAD // _CHUNK, _CHUNK)
    dst2d = dst_p.reshape(_EPAD // _CHUNK, _CHUNK)
    zrows = jnp.zeros((_RPT, _H), f32)
    zdeg = jnp.zeros((_RPT,), f32)

    xl, xr = pl.pallas_call(
        _proj_body,
        grid=(_N // _BN,),
        in_specs=[pl.BlockSpec((_BN, _D), lambda i: (i, 0)),
                  pl.BlockSpec((_D, _H), lambda i: (0, 0)),
                  pl.BlockSpec((_D, _H), lambda i: (0, 0))],
        out_specs=[pl.BlockSpec((_BN, _H), lambda i: (i, 0)),
                   pl.BlockSpec((_BN, _H), lambda i: (i, 0))],
        out_shape=[jax.ShapeDtypeStruct((_N, _H), f32),
                   jax.ShapeDtypeStruct((_N, _H), f32)],
    )(x, W_l1, W_r1)

    s1p, degp = _seg_with_deg(xl, src2d, dst2d, zrows, zdeg)
    s1p = s1p[:, :_N, :]
    degp = degp[:, :_N, None]

    h, dinv = pl.pallas_call(
        _layer1_body,
        grid=(_N // _BN,),
        in_specs=[pl.BlockSpec((2, _BN, _H), lambda i: (0, i, 0)),
                  pl.BlockSpec((2, _BN, 1), lambda i: (0, i, 0)),
                  pl.BlockSpec((_BN, _H), lambda i: (i, 0)),
                  pl.BlockSpec((1, _H), lambda i: (0, 0))],
        out_specs=[pl.BlockSpec((_BN, _H), lambda i: (i, 0)),
                   pl.BlockSpec((_BN, 1), lambda i: (i, 0))],
        out_shape=[jax.ShapeDtypeStruct((_N, _H), f32),
                   jax.ShapeDtypeStruct((_N, 1), f32)],
    )(s1p, degp, xr, b_l1.reshape(1, _H))

    s2p = _seg_no_deg(h, src2d, dst2d, zrows, zdeg)
    s2p = s2p[:, :_N, :]

    out = pl.pallas_call(
        _layer2_body,
        grid=(_N // _BN,),
        in_specs=[pl.BlockSpec((2, _BN, _H), lambda i: (0, i, 0)),
                  pl.BlockSpec((_BN, 1), lambda i: (i, 0)),
                  pl.BlockSpec((_BN, _H), lambda i: (i, 0)),
                  pl.BlockSpec((_H, _C), lambda i: (0, 0)),
                  pl.BlockSpec((1, _C), lambda i: (0, 0)),
                  pl.BlockSpec((_H, _C), lambda i: (0, 0))],
        out_specs=pl.BlockSpec((_BN, _C), lambda i: (i, 0)),
        out_shape=jax.ShapeDtypeStruct((_N, _C), f32),
    )(s2p, dinv, h, W_l2, b_l2.reshape(1, _C), W_r2)
    return out


# speedup vs baseline: 20.9388x; 1.5504x over previous
import functools

import jax
import jax.numpy as jnp
from jax import lax
from jax.experimental import pallas as pl
from jax.experimental.pallas import tpu as pltpu
from jax.experimental.pallas import tpu_sc as plsc

_N = 10000
_E = 320000
_D = 128
_H = 16
_C = 300

_NC = 2
_NS = 16
_NW = _NC * _NS
_CHUNK = 128
_TOTCH = _E // _CHUNK
_BASECH = _TOTCH // _NW
_EXTRA = _TOTCH - _BASECH * _NW
_MAXCH = _BASECH + 1
_NPAD = _NW * 320
_RPT = _NPAD // _NS
_NB = 4


def _seg_body(with_deg, xl_hbm, src_hbm, dst_hbm, ones_hbm, zdeg_hbm, *refs):
    if with_deg:
        (s_out, deg_out, s_sh, deg_sh, src_v, dst_v, rows_v, ones_v,
         tmp_rows, tmp_deg, gsem, ssem, dsem) = refs
    else:
        (s_out, s_sh, src_v, dst_v, rows_v, tmp_rows, gsem, ssem, dsem) = refs
    c = lax.axis_index("c")
    s = lax.axis_index("s")
    wid = c * _NS + s
    nch = jnp.where(wid < _EXTRA, _BASECH + 1, _BASECH)
    base = wid * _BASECH + jnp.minimum(wid, _EXTRA)

    def zrow(i, carry):
        tmp_rows[i] = jnp.zeros((_H,), jnp.float32)
        return carry
    lax.fori_loop(0, _RPT, zrow, 0)
    pltpu.sync_copy(tmp_rows, s_sh.at[pl.ds(s * _RPT, _RPT)])
    if with_deg:
        pltpu.sync_copy(zdeg_hbm, tmp_deg)
        pltpu.sync_copy(tmp_deg, deg_sh.at[pl.ds(s * _RPT, _RPT)])
        pltpu.sync_copy(ones_hbm, ones_v)

    pltpu.sync_copy(src_hbm.at[pl.ds(base, _BASECH)],
                    src_v.at[pl.ds(0, _BASECH)])
    pltpu.sync_copy(dst_hbm.at[pl.ds(base, _BASECH)],
                    dst_v.at[pl.ds(0, _BASECH)])

    @pl.when(wid < _EXTRA)
    def _stage_extra():
        pltpu.sync_copy(src_hbm.at[pl.ds(base + _BASECH, 1)],
                        src_v.at[pl.ds(_BASECH, 1)])
        pltpu.sync_copy(dst_hbm.at[pl.ds(base + _BASECH, 1)],
                        dst_v.at[pl.ds(_BASECH, 1)])

    plsc.subcore_barrier()

    for p in range(_NB - 1):
        pltpu.async_copy(xl_hbm.at[src_v.at[p]], rows_v.at[p], gsem.at[p])

    def body(j, carry):
        b = lax.rem(j, _NB)
        ahead = j + (_NB - 1)
        fb = lax.rem(ahead, _NB)

        @pl.when(ahead < nch)
        def _fire_ahead():
            @pl.when(j >= 1)
            def _drain_scatter():
                pltpu.make_async_copy(
                    rows_v.at[fb], s_sh.at[dst_v.at[j]], ssem.at[fb]).wait()
            pltpu.async_copy(xl_hbm.at[src_v.at[ahead]], rows_v.at[fb],
                             gsem.at[fb])

        pltpu.make_async_copy(xl_hbm.at[src_v.at[j]], rows_v.at[b],
                              gsem.at[b]).wait()
        pltpu.async_copy(rows_v.at[b], s_sh.at[dst_v.at[j]], ssem.at[b],
                         add=True)
        if with_deg:
            @pl.when(j >= 1)
            def _drain_deg():
                pltpu.make_async_copy(ones_v, deg_sh.at[dst_v.at[j]],
                                      dsem).wait()
            pltpu.async_copy(ones_v, deg_sh.at[dst_v.at[j]], dsem, add=True)
        return carry

    lax.fori_loop(0, nch, body, 0)
    for b in range(_NB):
        pltpu.make_async_copy(rows_v.at[b], s_sh.at[dst_v.at[0]],
                              ssem.at[b]).wait()
    if with_deg:
        pltpu.make_async_copy(ones_v, deg_sh.at[dst_v.at[0]], dsem).wait()
    plsc.subcore_barrier()

    pltpu.sync_copy(s_sh.at[pl.ds(s * _RPT, _RPT)], tmp_rows)
    pltpu.sync_copy(tmp_rows, s_out.at[c, pl.ds(s * _RPT, _RPT)])
    if with_deg:
        pltpu.sync_copy(deg_sh.at[pl.ds(s * _RPT, _RPT)], tmp_deg)
        pltpu.sync_copy(tmp_deg, deg_out.at[c, pl.ds(s * _RPT, _RPT)])


def _make_seg_kernel(with_deg):
    mesh = plsc.VectorSubcoreMesh(core_axis_name="c", subcore_axis_name="s")
    if with_deg:
        out_type = (jax.ShapeDtypeStruct((_NC, _NPAD, _H), jnp.float32),
                    jax.ShapeDtypeStruct((_NC, _NPAD, 1), jnp.float32))
    else:
        out_type = jax.ShapeDtypeStruct((_NC, _NPAD, _H), jnp.float32)
    scratch = [pltpu.VMEM_SHARED((_NPAD, _H), jnp.float32)]
    if with_deg:
        scratch.append(pltpu.VMEM_SHARED((_NPAD, 1), jnp.float32))
    scratch += [
        pltpu.VMEM((_MAXCH, _CHUNK), jnp.int32),
        pltpu.VMEM((_MAXCH, _CHUNK), jnp.int32),
        pltpu.VMEM((_NB, _CHUNK, _H), jnp.float32),
    ]
    if with_deg:
        scratch.append(pltpu.VMEM((_CHUNK, 1), jnp.float32))
    scratch.append(pltpu.VMEM((_RPT, _H), jnp.float32))
    if with_deg:
        scratch.append(pltpu.VMEM((_RPT, 1), jnp.float32))
    scratch += [pltpu.SemaphoreType.DMA((_NB,)),
                pltpu.SemaphoreType.DMA((_NB,)),
                pltpu.SemaphoreType.DMA]
    return pl.kernel(functools.partial(_seg_body, with_deg),
                     out_type=out_type, mesh=mesh, scratch_types=scratch,
                     compiler_params=pltpu.CompilerParams(
                         use_tc_tiling_on_sc=False))


_seg_with_deg = _make_seg_kernel(True)
_seg_no_deg = _make_seg_kernel(False)


def _proj_body(x_ref, wl_ref, wr_ref, xl_ref, xr_ref):
    x = x_ref[...]
    xl_ref[...] = jnp.dot(x, wl_ref[...], preferred_element_type=jnp.float32)
    xr_ref[...] = jnp.dot(x, wr_ref[...], preferred_element_type=jnp.float32)


def _layer1_body(s1_ref, deg_ref, xr_ref, b_ref, h_ref, dinv_ref):
    deg = deg_ref[0] + deg_ref[1]
    dinv = 1.0 / jnp.maximum(deg, 1.0)
    s1 = s1_ref[0] + s1_ref[1]
    h_ref[...] = jnp.maximum(s1 * dinv + b_ref[...] + xr_ref[...], 0.0)
    dinv_ref[...] = dinv


def _layer2_body(s2_ref, dinv_ref, h_ref, wl_ref, b_ref, wr_ref, o_ref):
    agg = (s2_ref[0] + s2_ref[1]) * dinv_ref[...]
    z = (jnp.dot(agg, wl_ref[...], preferred_element_type=jnp.float32)
         + b_ref[...]
         + jnp.dot(h_ref[...], wr_ref[...], preferred_element_type=jnp.float32))
    m = jnp.max(z, axis=1, keepdims=True)
    lse = jnp.log(jnp.sum(jnp.exp(z - m), axis=1, keepdims=True)) + m
    o_ref[...] = z - lse


_BN = 2000


def kernel(x, edge_index, W_l1, b_l1, W_r1, W_l2, b_l2, W_r2):
    f32 = jnp.float32
    src2d = edge_index[0].reshape(_TOTCH, _CHUNK)
    dst2d = edge_index[1].reshape(_TOTCH, _CHUNK)

    xl, xr = pl.pallas_call(
        _proj_body,
        grid=(_N // _BN,),
        in_specs=[pl.BlockSpec((_BN, _D), lambda i: (i, 0)),
                  pl.BlockSpec((_D, _H), lambda i: (0, 0)),
                  pl.BlockSpec((_D, _H), lambda i: (0, 0))],
        out_specs=[pl.BlockSpec((_BN, _H), lambda i: (i, 0)),
                   pl.BlockSpec((_BN, _H), lambda i: (i, 0))],
        out_shape=[jax.ShapeDtypeStruct((_N, _H), f32),
                   jax.ShapeDtypeStruct((_N, _H), f32)],
    )(x, W_l1, W_r1)

    ones_c = jnp.ones((_CHUNK, 1), f32)
    zdeg_c = jnp.zeros((_RPT, 1), f32)

    s1p, degp = _seg_with_deg(xl, src2d, dst2d, ones_c, zdeg_c)

    h, dinv = pl.pallas_call(
        _layer1_body,
        grid=(_N // _BN,),
        in_specs=[pl.BlockSpec((2, _BN, _H), lambda i: (0, i, 0)),
                  pl.BlockSpec((2, _BN, 1), lambda i: (0, i, 0)),
                  pl.BlockSpec((_BN, _H), lambda i: (i, 0)),
                  pl.BlockSpec((1, _H), lambda i: (0, 0))],
        out_specs=[pl.BlockSpec((_BN, _H), lambda i: (i, 0)),
                   pl.BlockSpec((_BN, 1), lambda i: (i, 0))],
        out_shape=[jax.ShapeDtypeStruct((_N, _H), f32),
                   jax.ShapeDtypeStruct((_N, 1), f32)],
    )(s1p, degp, xr, b_l1.reshape(1, _H))

    s2p = _seg_no_deg(h, src2d, dst2d, ones_c, zdeg_c)

    out = pl.pallas_call(
        _layer2_body,
        grid=(_N // _BN,),
        in_specs=[pl.BlockSpec((2, _BN, _H), lambda i: (0, i, 0)),
                  pl.BlockSpec((_BN, 1), lambda i: (i, 0)),
                  pl.BlockSpec((_BN, _H), lambda i: (i, 0)),
                  pl.BlockSpec((_H, _C), lambda i: (0, 0)),
                  pl.BlockSpec((1, _C), lambda i: (0, 0)),
                  pl.BlockSpec((_H, _C), lambda i: (0, 0))],
        out_specs=pl.BlockSpec((_BN, _C), lambda i: (i, 0)),
        out_shape=jax.ShapeDtypeStruct((_N, _C), f32),
    )(s2p, dinv, h, W_l2, b_l2.reshape(1, _C), W_r2)
    return out


# speedup vs baseline: 21.2733x; 1.0160x over previous
import functools

import jax
import jax.numpy as jnp
from jax import lax
from jax.experimental import pallas as pl
from jax.experimental.pallas import tpu as pltpu
from jax.experimental.pallas import tpu_sc as plsc

_N = 10000
_E = 320000
_D = 128
_H = 16
_C = 300

_NC = 2
_NS = 16
_NW = _NC * _NS
_CHUNK = 128
_TOTCH = _E // _CHUNK
_BASECH = _TOTCH // _NW
_EXTRA = _TOTCH - _BASECH * _NW
_MAXCH = _BASECH + 1
_NPAD = _NW * 320
_RPT = _NPAD // _NS
_NB = 4


def _seg_body(with_deg, xl_hbm, src_hbm, dst_hbm, ones_hbm, zdeg_hbm,
              zrows_hbm, *refs):
    if with_deg:
        (s_out, deg_out, s_sh, deg_sh, src_v, dst_v, rows_v, ones_v,
         tmp_rows, tmp_deg, gsem, ssem, dsem) = refs
    else:
        (s_out, s_sh, src_v, dst_v, rows_v, tmp_rows, gsem, ssem, dsem) = refs
    c = lax.axis_index("c")
    s = lax.axis_index("s")
    wid = c * _NS + s
    nch = jnp.where(wid < _EXTRA, _BASECH + 1, _BASECH)
    base = wid * _BASECH + jnp.minimum(wid, _EXTRA)

    pltpu.sync_copy(zrows_hbm, tmp_rows)
    pltpu.sync_copy(tmp_rows, s_sh.at[pl.ds(s * _RPT, _RPT)])
    if with_deg:
        pltpu.sync_copy(zdeg_hbm, tmp_deg)
        pltpu.sync_copy(tmp_deg, deg_sh.at[pl.ds(s * _RPT, _RPT)])
        pltpu.sync_copy(ones_hbm, ones_v)

    pltpu.sync_copy(src_hbm.at[pl.ds(base, _BASECH)],
                    src_v.at[pl.ds(0, _BASECH)])
    pltpu.sync_copy(dst_hbm.at[pl.ds(base, _BASECH)],
                    dst_v.at[pl.ds(0, _BASECH)])

    @pl.when(wid < _EXTRA)
    def _stage_extra():
        pltpu.sync_copy(src_hbm.at[pl.ds(base + _BASECH, 1)],
                        src_v.at[pl.ds(_BASECH, 1)])
        pltpu.sync_copy(dst_hbm.at[pl.ds(base + _BASECH, 1)],
                        dst_v.at[pl.ds(_BASECH, 1)])

    plsc.subcore_barrier()

    for p in range(_NB - 1):
        pltpu.async_copy(xl_hbm.at[src_v.at[p]], rows_v.at[p], gsem.at[p])

    def body(j, carry):
        b = lax.rem(j, _NB)
        ahead = j + (_NB - 1)
        fb = lax.rem(ahead, _NB)

        @pl.when(ahead < nch)
        def _fire_ahead():
            @pl.when(j >= 1)
            def _drain_scatter():
                pltpu.make_async_copy(
                    rows_v.at[fb], s_sh.at[dst_v.at[j]], ssem.at[fb]).wait()
            pltpu.async_copy(xl_hbm.at[src_v.at[ahead]], rows_v.at[fb],
                             gsem.at[fb])

        @pl.when(j < nch)
        def _consume():
            pltpu.make_async_copy(xl_hbm.at[src_v.at[j]], rows_v.at[b],
                                  gsem.at[b]).wait()
            pltpu.async_copy(rows_v.at[b], s_sh.at[dst_v.at[j]], ssem.at[b],
                             add=True)
            if with_deg:
                @pl.when(j >= 1)
                def _drain_deg():
                    pltpu.make_async_copy(ones_v, deg_sh.at[dst_v.at[j]],
                                          dsem).wait()
                pltpu.async_copy(ones_v, deg_sh.at[dst_v.at[j]], dsem,
                                 add=True)
        return carry

    lax.fori_loop(0, _MAXCH, body, 0)
    for b in range(_NB):
        pltpu.make_async_copy(rows_v.at[b], s_sh.at[dst_v.at[0]],
                              ssem.at[b]).wait()
    if with_deg:
        pltpu.make_async_copy(ones_v, deg_sh.at[dst_v.at[0]], dsem).wait()
    plsc.subcore_barrier()

    pltpu.sync_copy(s_sh.at[pl.ds(s * _RPT, _RPT)], tmp_rows)
    pltpu.sync_copy(tmp_rows, s_out.at[c, pl.ds(s * _RPT, _RPT)])
    if with_deg:
        pltpu.sync_copy(deg_sh.at[pl.ds(s * _RPT, _RPT)], tmp_deg)
        pltpu.sync_copy(tmp_deg, deg_out.at[c, pl.ds(s * _RPT, _RPT)])


def _make_seg_kernel(with_deg):
    mesh = plsc.VectorSubcoreMesh(core_axis_name="c", subcore_axis_name="s")
    if with_deg:
        out_type = (jax.ShapeDtypeStruct((_NC, _NPAD, _H), jnp.float32),
                    jax.ShapeDtypeStruct((_NC, _NPAD), jnp.float32))
    else:
        out_type = jax.ShapeDtypeStruct((_NC, _NPAD, _H), jnp.float32)
    scratch = [pltpu.VMEM_SHARED((_NPAD, _H), jnp.float32)]
    if with_deg:
        scratch.append(pltpu.VMEM_SHARED((_NPAD,), jnp.float32))
    scratch += [
        pltpu.VMEM((_MAXCH, _CHUNK), jnp.int32),
        pltpu.VMEM((_MAXCH, _CHUNK), jnp.int32),
        pltpu.VMEM((_NB, _CHUNK, _H), jnp.float32),
    ]
    if with_deg:
        scratch.append(pltpu.VMEM((_CHUNK,), jnp.float32))
    scratch.append(pltpu.VMEM((_RPT, _H), jnp.float32))
    if with_deg:
        scratch.append(pltpu.VMEM((_RPT,), jnp.float32))
    scratch += [pltpu.SemaphoreType.DMA((_NB,)),
                pltpu.SemaphoreType.DMA((_NB,)),
                pltpu.SemaphoreType.DMA]
    return pl.kernel(functools.partial(_seg_body, with_deg),
                     out_type=out_type, mesh=mesh, scratch_types=scratch,
                     compiler_params=pltpu.CompilerParams(
                         use_tc_tiling_on_sc=False))


_seg_with_deg = _make_seg_kernel(True)
_seg_no_deg = _make_seg_kernel(False)


def _proj_body(x_ref, wl_ref, wr_ref, xl_ref, xr_ref):
    x = x_ref[...]
    xl_ref[...] = jnp.dot(x, wl_ref[...], preferred_element_type=jnp.float32)
    xr_ref[...] = jnp.dot(x, wr_ref[...], preferred_element_type=jnp.float32)


def _layer1_body(s1_ref, deg_ref, xr_ref, b_ref, h_ref, dinv_ref):
    deg = deg_ref[0] + deg_ref[1]
    dinv = 1.0 / jnp.maximum(deg, 1.0)
    s1 = s1_ref[0] + s1_ref[1]
    h_ref[...] = jnp.maximum(s1 * dinv + b_ref[...] + xr_ref[...], 0.0)
    dinv_ref[...] = dinv


def _layer2_body(s2_ref, dinv_ref, h_ref, wl_ref, b_ref, wr_ref, o_ref):
    agg = (s2_ref[0] + s2_ref[1]) * dinv_ref[...]
    z = (jnp.dot(agg, wl_ref[...], preferred_element_type=jnp.float32)
         + b_ref[...]
         + jnp.dot(h_ref[...], wr_ref[...], preferred_element_type=jnp.float32))
    m = jnp.max(z, axis=1, keepdims=True)
    lse = jnp.log(jnp.sum(jnp.exp(z - m), axis=1, keepdims=True)) + m
    o_ref[...] = z - lse


_BN = 2000


def kernel(x, edge_index, W_l1, b_l1, W_r1, W_l2, b_l2, W_r2):
    f32 = jnp.float32
    src2d = edge_index[0].reshape(_TOTCH, _CHUNK)
    dst2d = edge_index[1].reshape(_TOTCH, _CHUNK)

    xl, xr = pl.pallas_call(
        _proj_body,
        grid=(_N // _BN,),
        in_specs=[pl.BlockSpec((_BN, _D), lambda i: (i, 0)),
                  pl.BlockSpec((_D, _H), lambda i: (0, 0)),
                  pl.BlockSpec((_D, _H), lambda i: (0, 0))],
        out_specs=[pl.BlockSpec((_BN, _H), lambda i: (i, 0)),
                   pl.BlockSpec((_BN, _H), lambda i: (i, 0))],
        out_shape=[jax.ShapeDtypeStruct((_N, _H), f32),
                   jax.ShapeDtypeStruct((_N, _H), f32)],
    )(x, W_l1, W_r1)

    ones_c = jnp.ones((_CHUNK,), f32)
    zdeg_c = jnp.zeros((_RPT,), f32)
    zrows_c = jnp.zeros((_RPT, _H), f32)

    s1p, degp = _seg_with_deg(xl, src2d, dst2d, ones_c, zdeg_c, zrows_c)
    degp = degp[:, :_N, None]

    h, dinv = pl.pallas_call(
        _layer1_body,
        grid=(_N // _BN,),
        in_specs=[pl.BlockSpec((2, _BN, _H), lambda i: (0, i, 0)),
                  pl.BlockSpec((2, _BN, 1), lambda i: (0, i, 0)),
                  pl.BlockSpec((_BN, _H), lambda i: (i, 0)),
                  pl.BlockSpec((1, _H), lambda i: (0, 0))],
        out_specs=[pl.BlockSpec((_BN, _H), lambda i: (i, 0)),
                   pl.BlockSpec((_BN, 1), lambda i: (i, 0))],
        out_shape=[jax.ShapeDtypeStruct((_N, _H), f32),
                   jax.ShapeDtypeStruct((_N, 1), f32)],
    )(s1p, degp, xr, b_l1.reshape(1, _H))

    s2p = _seg_no_deg(h, src2d, dst2d, ones_c, zdeg_c, zrows_c)

    out = pl.pallas_call(
        _layer2_body,
        grid=(_N // _BN,),
        in_specs=[pl.BlockSpec((2, _BN, _H), lambda i: (0, i, 0)),
                  pl.BlockSpec((_BN, 1), lambda i: (i, 0)),
                  pl.BlockSpec((_BN, _H), lambda i: (i, 0)),
                  pl.BlockSpec((_H, _C), lambda i: (0, 0)),
                  pl.BlockSpec((1, _C), lambda i: (0, 0)),
                  pl.BlockSpec((_H, _C), lambda i: (0, 0))],
        out_specs=pl.BlockSpec((_BN, _C), lambda i: (i, 0)),
        out_shape=jax.ShapeDtypeStruct((_N, _C), f32),
    )(s2p, dinv, h, W_l2, b_l2.reshape(1, _C), W_r2)
    return out


# speedup vs baseline: 22.8224x; 1.0728x over previous
import functools

import jax
import jax.numpy as jnp
from jax import lax
from jax.experimental import pallas as pl
from jax.experimental.pallas import tpu as pltpu
from jax.experimental.pallas import tpu_sc as plsc

_N = 10000
_E = 320000
_D = 128
_H = 16
_C = 300

_NC = 2
_NS = 16
_NW = _NC * _NS
_CHUNK = 128
_TOTCH = _E // _CHUNK
_BASECH = _TOTCH // _NW
_EXTRA = _TOTCH - _BASECH * _NW
_MAXCH = _BASECH + 1
_NPAD = _NW * 320
_RPT = _NPAD // _NS
_NB = 4


def _seg_body(with_deg, xl_hbm, src_hbm, dst_hbm, ones_hbm, zdeg_hbm,
              zrows_hbm, *refs):
    if with_deg:
        (s_out, deg_out, s_sh, deg_sh, src_v, dst_v, rows_v, ones_v,
         tmp_rows, tmp_deg, gsem, ssem, dsem) = refs
    else:
        (s_out, s_sh, src_v, dst_v, rows_v, tmp_rows, gsem, ssem, dsem) = refs
    c = lax.axis_index("c")
    s = lax.axis_index("s")
    wid = c * _NS + s
    nch = jnp.where(wid < _EXTRA, _BASECH + 1, _BASECH)
    base = wid * _BASECH + jnp.minimum(wid, _EXTRA)

    pltpu.sync_copy(zrows_hbm, tmp_rows)
    pltpu.sync_copy(tmp_rows, s_sh.at[pl.ds(s * _RPT, _RPT)])
    if with_deg:
        pltpu.sync_copy(zdeg_hbm, tmp_deg)
        pltpu.sync_copy(tmp_deg, deg_sh.at[pl.ds(s * _RPT, _RPT)])
        pltpu.sync_copy(ones_hbm, ones_v)

    pltpu.sync_copy(src_hbm.at[pl.ds(base, _BASECH)],
                    src_v.at[pl.ds(0, _BASECH)])
    pltpu.sync_copy(dst_hbm.at[pl.ds(base, _BASECH)],
                    dst_v.at[pl.ds(0, _BASECH)])

    @pl.when(wid < _EXTRA)
    def _stage_extra():
        pltpu.sync_copy(src_hbm.at[pl.ds(base + _BASECH, 1)],
                        src_v.at[pl.ds(_BASECH, 1)])
        pltpu.sync_copy(dst_hbm.at[pl.ds(base + _BASECH, 1)],
                        dst_v.at[pl.ds(_BASECH, 1)])

    plsc.subcore_barrier()

    for p in range(_NB - 1):
        pltpu.async_copy(xl_hbm.at[src_v.at[p]], rows_v.at[p], gsem.at[p])

    def body(j, carry):
        b = lax.rem(j, _NB)
        ahead = j + (_NB - 1)
        fb = lax.rem(ahead, _NB)

        @pl.when(ahead < nch)
        def _fire_ahead():
            @pl.when(j >= 1)
            def _drain_scatter():
                pltpu.make_async_copy(
                    rows_v.at[fb], s_sh.at[dst_v.at[j]], ssem.at[fb]).wait()
            pltpu.async_copy(xl_hbm.at[src_v.at[ahead]], rows_v.at[fb],
                             gsem.at[fb])

        @pl.when(j < nch)
        def _consume():
            pltpu.make_async_copy(xl_hbm.at[src_v.at[j]], rows_v.at[b],
                                  gsem.at[b]).wait()
            pltpu.async_copy(rows_v.at[b], s_sh.at[dst_v.at[j]], ssem.at[b],
                             add=True)
            if with_deg:
                @pl.when(j >= 1)
                def _drain_deg():
                    pltpu.make_async_copy(ones_v, deg_sh.at[dst_v.at[j]],
                                          dsem).wait()
                pltpu.async_copy(ones_v, deg_sh.at[dst_v.at[j]], dsem,
                                 add=True)
        return carry

    lax.fori_loop(0, _MAXCH, body, 0)
    for b in range(_NB):
        pltpu.make_async_copy(rows_v.at[b], s_sh.at[dst_v.at[0]],
                              ssem.at[b]).wait()
    if with_deg:
        pltpu.make_async_copy(ones_v, deg_sh.at[dst_v.at[0]], dsem).wait()
    plsc.subcore_barrier()

    pltpu.sync_copy(s_sh.at[pl.ds(s * _RPT, _RPT)], tmp_rows)
    pltpu.sync_copy(tmp_rows, s_out.at[c, pl.ds(s * _RPT, _RPT)])
    if with_deg:
        pltpu.sync_copy(deg_sh.at[pl.ds(s * _RPT, _RPT)], tmp_deg)
        pltpu.sync_copy(tmp_deg, deg_out.at[c, pl.ds(s * _RPT, _RPT)])


def _make_seg_kernel(with_deg):
    mesh = plsc.VectorSubcoreMesh(core_axis_name="c", subcore_axis_name="s")
    if with_deg:
        out_type = (jax.ShapeDtypeStruct((_NC, _NPAD, _H), jnp.float32),
                    jax.ShapeDtypeStruct((_NC, _NPAD), jnp.float32))
    else:
        out_type = jax.ShapeDtypeStruct((_NC, _NPAD, _H), jnp.float32)
    scratch = [pltpu.VMEM_SHARED((_NPAD, _H), jnp.float32)]
    if with_deg:
        scratch.append(pltpu.VMEM_SHARED((_NPAD,), jnp.float32))
    scratch += [
        pltpu.VMEM((_MAXCH, _CHUNK), jnp.int32),
        pltpu.VMEM((_MAXCH, _CHUNK), jnp.int32),
        pltpu.VMEM((_NB, _CHUNK, _H), jnp.float32),
    ]
    if with_deg:
        scratch.append(pltpu.VMEM((_CHUNK,), jnp.float32))
    scratch.append(pltpu.VMEM((_RPT, _H), jnp.float32))
    if with_deg:
        scratch.append(pltpu.VMEM((_RPT,), jnp.float32))
    scratch += [pltpu.SemaphoreType.DMA((_NB,)),
                pltpu.SemaphoreType.DMA((_NB,)),
                pltpu.SemaphoreType.DMA]
    return pl.kernel(functools.partial(_seg_body, with_deg),
                     out_type=out_type, mesh=mesh, scratch_types=scratch,
                     compiler_params=pltpu.CompilerParams(
                         use_tc_tiling_on_sc=False))


_seg_with_deg = _make_seg_kernel(True)
_seg_no_deg = _make_seg_kernel(False)


def _proj_body(x_ref, wl_ref, wr_ref, xl_ref, xr_ref):
    x = x_ref[...]
    xl_ref[...] = jnp.dot(x, wl_ref[...], preferred_element_type=jnp.float32)
    xr_ref[...] = jnp.dot(x, wr_ref[...], preferred_element_type=jnp.float32)


def _layer1_body(s1_ref, deg_ref, xr_ref, b_ref, h_ref, dinv_ref):
    deg = deg_ref[0] + deg_ref[1]
    dinv = 1.0 / jnp.maximum(deg, 1.0)
    s1 = s1_ref[0] + s1_ref[1]
    h_ref[...] = jnp.maximum(s1 * dinv + b_ref[...] + xr_ref[...], 0.0)
    dinv_ref[...] = dinv


def _layer2_body(s2_ref, dinv_ref, h_ref, wcat_ref, b_ref, o_ref):
    agg = (s2_ref[0, :_N] + s2_ref[1, :_N]) * dinv_ref[...]
    cat_t = jnp.concatenate([agg, h_ref[...]], axis=1).T
    z = (jnp.dot(wcat_ref[...], cat_t, preferred_element_type=jnp.float32)
         + b_ref[...])
    m = jnp.max(z, axis=0, keepdims=True)
    lse = jnp.log(jnp.sum(jnp.exp(z - m), axis=0, keepdims=True)) + m
    o_ref[...] = z - lse


_BN = 2000


def kernel(x, edge_index, W_l1, b_l1, W_r1, W_l2, b_l2, W_r2):
    f32 = jnp.float32
    src2d = edge_index[0].reshape(_TOTCH, _CHUNK)
    dst2d = edge_index[1].reshape(_TOTCH, _CHUNK)

    xl, xr = pl.pallas_call(
        _proj_body,
        grid=(_N // _BN,),
        in_specs=[pl.BlockSpec((_BN, _D), lambda i: (i, 0)),
                  pl.BlockSpec((_D, _H), lambda i: (0, 0)),
                  pl.BlockSpec((_D, _H), lambda i: (0, 0))],
        out_specs=[pl.BlockSpec((_BN, _H), lambda i: (i, 0)),
                   pl.BlockSpec((_BN, _H), lambda i: (i, 0))],
        out_shape=[jax.ShapeDtypeStruct((_N, _H), f32),
                   jax.ShapeDtypeStruct((_N, _H), f32)],
    )(x, W_l1, W_r1)

    ones_c = jnp.ones((_CHUNK,), f32)
    zdeg_c = jnp.zeros((_RPT,), f32)
    zrows_c = jnp.zeros((_RPT, _H), f32)

    s1p, degp = _seg_with_deg(xl, src2d, dst2d, ones_c, zdeg_c, zrows_c)
    degp = degp[:, :_N, None]

    h, dinv = pl.pallas_call(
        _layer1_body,
        grid=(_N // _BN,),
        in_specs=[pl.BlockSpec((2, _BN, _H), lambda i: (0, i, 0)),
                  pl.BlockSpec((2, _BN, 1), lambda i: (0, i, 0)),
                  pl.BlockSpec((_BN, _H), lambda i: (i, 0)),
                  pl.BlockSpec((1, _H), lambda i: (0, 0))],
        out_specs=[pl.BlockSpec((_BN, _H), lambda i: (i, 0)),
                   pl.BlockSpec((_BN, 1), lambda i: (i, 0))],
        out_shape=[jax.ShapeDtypeStruct((_N, _H), f32),
                   jax.ShapeDtypeStruct((_N, 1), f32)],
    )(s1p, degp, xr, b_l1.reshape(1, _H))

    s2p = _seg_no_deg(h, src2d, dst2d, ones_c, zdeg_c, zrows_c)

    w_cat_t = jnp.concatenate([W_l2, W_r2], axis=0).T
    out_t = pl.pallas_call(
        _layer2_body,
        out_shape=jax.ShapeDtypeStruct((_C, _N), f32),
    )(s2p, dinv, h, w_cat_t, b_l2.reshape(_C, 1))
    return out_t.T


# speedup vs baseline: 24.4551x; 1.0715x over previous
import functools

import jax
import jax.numpy as jnp
from jax import lax
from jax.experimental import pallas as pl
from jax.experimental.pallas import tpu as pltpu
from jax.experimental.pallas import tpu_sc as plsc

_N = 10000
_E = 320000
_D = 128
_H = 16
_C = 300

_NC = 2
_NS = 16
_NW = _NC * _NS
_CHUNK = 128
_TOTCH = _E // _CHUNK
_BASECH = _TOTCH // _NW
_EXTRA = _TOTCH - _BASECH * _NW
_MAXCH = _BASECH + 1
_NPAD = _NW * 320
_RPT = _NPAD // _NS
_NB = 6
_DLAG = 4


def _seg_body(with_deg, xl_hbm, src_hbm, dst_hbm, ones_hbm, zdeg_hbm,
              zrows_hbm, *refs):
    if with_deg:
        (s_out, deg_out, s_sh, deg_sh, src_v, dst_v, rows_v, ones_v,
         tmp_rows, tmp_deg, gsem, ssem, dsem) = refs
    else:
        (s_out, s_sh, src_v, dst_v, rows_v, tmp_rows, gsem, ssem, dsem) = refs
    c = lax.axis_index("c")
    s = lax.axis_index("s")
    wid = c * _NS + s
    nch = jnp.where(wid < _EXTRA, _BASECH + 1, _BASECH)
    base = wid * _BASECH + jnp.minimum(wid, _EXTRA)

    pltpu.sync_copy(zrows_hbm, tmp_rows)
    pltpu.sync_copy(tmp_rows, s_sh.at[pl.ds(s * _RPT, _RPT)])
    if with_deg:
        pltpu.sync_copy(zdeg_hbm, tmp_deg)
        pltpu.sync_copy(tmp_deg, deg_sh.at[pl.ds(s * _RPT, _RPT)])
        pltpu.sync_copy(ones_hbm, ones_v)

    pltpu.sync_copy(src_hbm.at[pl.ds(base, _BASECH)],
                    src_v.at[pl.ds(0, _BASECH)])
    pltpu.sync_copy(dst_hbm.at[pl.ds(base, _BASECH)],
                    dst_v.at[pl.ds(0, _BASECH)])

    @pl.when(wid < _EXTRA)
    def _stage_extra():
        pltpu.sync_copy(src_hbm.at[pl.ds(base + _BASECH, 1)],
                        src_v.at[pl.ds(_BASECH, 1)])
        pltpu.sync_copy(dst_hbm.at[pl.ds(base + _BASECH, 1)],
                        dst_v.at[pl.ds(_BASECH, 1)])

    plsc.subcore_barrier()

    for p in range(_NB - 1):
        pltpu.async_copy(xl_hbm.at[src_v.at[p]], rows_v.at[p], gsem.at[p])

    def body(j, carry):
        b = lax.rem(j, _NB)
        ahead = j + (_NB - 1)
        fb = lax.rem(ahead, _NB)

        @pl.when(ahead < nch)
        def _fire_ahead():
            @pl.when(j >= 1)
            def _drain_scatter():
                pltpu.make_async_copy(
                    rows_v.at[fb], s_sh.at[dst_v.at[j]], ssem.at[fb]).wait()
            pltpu.async_copy(xl_hbm.at[src_v.at[ahead]], rows_v.at[fb],
                             gsem.at[fb])

        @pl.when(j < nch)
        def _consume():
            pltpu.make_async_copy(xl_hbm.at[src_v.at[j]], rows_v.at[b],
                                  gsem.at[b]).wait()
            pltpu.async_copy(rows_v.at[b], s_sh.at[dst_v.at[j]], ssem.at[b],
                             add=True)
            if with_deg:
                @pl.when(j >= _DLAG)
                def _drain_deg():
                    pltpu.make_async_copy(ones_v, deg_sh.at[dst_v.at[j]],
                                          dsem).wait()
                pltpu.async_copy(ones_v, deg_sh.at[dst_v.at[j]], dsem,
                                 add=True)
        return carry

    lax.fori_loop(0, _MAXCH, body, 0)
    for b in range(_NB):
        pltpu.make_async_copy(rows_v.at[b], s_sh.at[dst_v.at[0]],
                              ssem.at[b]).wait()
    if with_deg:
        for _ in range(_DLAG):
            pltpu.make_async_copy(ones_v, deg_sh.at[dst_v.at[0]],
                                  dsem).wait()
    plsc.subcore_barrier()

    pltpu.sync_copy(s_sh.at[pl.ds(s * _RPT, _RPT)], tmp_rows)
    pltpu.sync_copy(tmp_rows, s_out.at[c, pl.ds(s * _RPT, _RPT)])
    if with_deg:
        pltpu.sync_copy(deg_sh.at[pl.ds(s * _RPT, _RPT)], tmp_deg)
        pltpu.sync_copy(tmp_deg, deg_out.at[c, pl.ds(s * _RPT, _RPT)])


def _make_seg_kernel(with_deg):
    mesh = plsc.VectorSubcoreMesh(core_axis_name="c", subcore_axis_name="s")
    if with_deg:
        out_type = (jax.ShapeDtypeStruct((_NC, _NPAD, _H), jnp.float32),
                    jax.ShapeDtypeStruct((_NC, _NPAD), jnp.float32))
    else:
        out_type = jax.ShapeDtypeStruct((_NC, _NPAD, _H), jnp.float32)
    scratch = [pltpu.VMEM_SHARED((_NPAD, _H), jnp.float32)]
    if with_deg:
        scratch.append(pltpu.VMEM_SHARED((_NPAD,), jnp.float32))
    scratch += [
        pltpu.VMEM((_MAXCH, _CHUNK), jnp.int32),
        pltpu.VMEM((_MAXCH, _CHUNK), jnp.int32),
        pltpu.VMEM((_NB, _CHUNK, _H), jnp.float32),
    ]
    if with_deg:
        scratch.append(pltpu.VMEM((_CHUNK,), jnp.float32))
    scratch.append(pltpu.VMEM((_RPT, _H), jnp.float32))
    if with_deg:
        scratch.append(pltpu.VMEM((_RPT,), jnp.float32))
    scratch += [pltpu.SemaphoreType.DMA((_NB,)),
                pltpu.SemaphoreType.DMA((_NB,)),
                pltpu.SemaphoreType.DMA]
    return pl.kernel(functools.partial(_seg_body, with_deg),
                     out_type=out_type, mesh=mesh, scratch_types=scratch,
                     compiler_params=pltpu.CompilerParams(
                         use_tc_tiling_on_sc=False))


_seg_with_deg = _make_seg_kernel(True)
_seg_no_deg = _make_seg_kernel(False)


def _proj_body(x_ref, wl_ref, wr_ref, xl_ref, xr_ref):
    x = x_ref[...]
    xl_ref[...] = jnp.dot(x, wl_ref[...], preferred_element_type=jnp.float32)
    xr_ref[...] = jnp.dot(x, wr_ref[...], preferred_element_type=jnp.float32)


def _layer1_body(s1_ref, deg_ref, xr_ref, b_ref, h_ref, dinv_ref):
    deg = deg_ref[0] + deg_ref[1]
    dinv = 1.0 / jnp.maximum(deg, 1.0)
    s1 = s1_ref[0] + s1_ref[1]
    h_ref[...] = jnp.maximum(s1 * dinv + b_ref[...] + xr_ref[...], 0.0)
    dinv_ref[...] = dinv


def _layer2_body(s2_ref, dinv_ref, h_ref, wcat_ref, b_ref, o_ref):
    agg = (s2_ref[0, :_N] + s2_ref[1, :_N]) * dinv_ref[...]
    cat_t = jnp.concatenate([agg, h_ref[...]], axis=1).T
    z = (jnp.dot(wcat_ref[...], cat_t, preferred_element_type=jnp.float32)
         + b_ref[...])
    m = jnp.max(z, axis=0, keepdims=True)
    lse = jnp.log(jnp.sum(jnp.exp(z - m), axis=0, keepdims=True)) + m
    o_ref[...] = z - lse


_BN = 2000


def kernel(x, edge_index, W_l1, b_l1, W_r1, W_l2, b_l2, W_r2):
    f32 = jnp.float32
    src2d = edge_index[0].reshape(_TOTCH, _CHUNK)
    dst2d = edge_index[1].reshape(_TOTCH, _CHUNK)

    xl, xr = pl.pallas_call(
        _proj_body,
        grid=(_N // _BN,),
        in_specs=[pl.BlockSpec((_BN, _D), lambda i: (i, 0)),
                  pl.BlockSpec((_D, _H), lambda i: (0, 0)),
                  pl.BlockSpec((_D, _H), lambda i: (0, 0))],
        out_specs=[pl.BlockSpec((_BN, _H), lambda i: (i, 0)),
                   pl.BlockSpec((_BN, _H), lambda i: (i, 0))],
        out_shape=[jax.ShapeDtypeStruct((_N, _H), f32),
                   jax.ShapeDtypeStruct((_N, _H), f32)],
    )(x, W_l1, W_r1)

    ones_c = jnp.ones((_CHUNK,), f32)
    zdeg_c = jnp.zeros((_RPT,), f32)
    zrows_c = jnp.zeros((_RPT, _H), f32)

    s1p, degp = _seg_with_deg(xl, src2d, dst2d, ones_c, zdeg_c, zrows_c)
    degp = degp[:, :_N, None]

    h, dinv = pl.pallas_call(
        _layer1_body,
        grid=(_N // _BN,),
        in_specs=[pl.BlockSpec((2, _BN, _H), lambda i: (0, i, 0)),
                  pl.BlockSpec((2, _BN, 1), lambda i: (0, i, 0)),
                  pl.BlockSpec((_BN, _H), lambda i: (i, 0)),
                  pl.BlockSpec((1, _H), lambda i: (0, 0))],
        out_specs=[pl.BlockSpec((_BN, _H), lambda i: (i, 0)),
                   pl.BlockSpec((_BN, 1), lambda i: (i, 0))],
        out_shape=[jax.ShapeDtypeStruct((_N, _H), f32),
                   jax.ShapeDtypeStruct((_N, 1), f32)],
    )(s1p, degp, xr, b_l1.reshape(1, _H))

    s2p = _seg_no_deg(h, src2d, dst2d, ones_c, zdeg_c, zrows_c)

    w_cat_t = jnp.concatenate([W_l2, W_r2], axis=0).T
    out_t = pl.pallas_call(
        _layer2_body,
        out_shape=jax.ShapeDtypeStruct((_C, _N), f32),
    )(s2p, dinv, h, w_cat_t, b_l2.reshape(_C, 1))
    return out_t.T


# speedup vs baseline: 25.0132x; 1.0228x over previous
import functools

import jax
import jax.numpy as jnp
from jax import lax
from jax.experimental import pallas as pl
from jax.experimental.pallas import tpu as pltpu
from jax.experimental.pallas import tpu_sc as plsc

_N = 10000
_E = 320000
_D = 128
_H = 16
_C = 300

_NC = 2
_NS = 16
_NW = _NC * _NS
_CHUNK = 128
_TOTCH = _E // _CHUNK
_BASECH = _TOTCH // _NW
_EXTRA = _TOTCH - _BASECH * _NW
_MAXCH = _BASECH + 1
_NPAD = _NW * 320
_RPT = _NPAD // _NS
_NB = 8
_DLAG = 4


def _seg_body(with_deg, xl_hbm, src_hbm, dst_hbm, ones_hbm, zdeg_hbm,
              zrows_hbm, *refs):
    if with_deg:
        (s_out, deg_out, s_sh, deg_sh, src_v, dst_v, rows_v, ones_v,
         tmp_rows, tmp_deg, gsem, ssem, dsem) = refs
    else:
        (s_out, s_sh, src_v, dst_v, rows_v, tmp_rows, gsem, ssem, dsem) = refs
    c = lax.axis_index("c")
    s = lax.axis_index("s")
    wid = c * _NS + s
    nch = jnp.where(wid < _EXTRA, _BASECH + 1, _BASECH)
    base = wid * _BASECH + jnp.minimum(wid, _EXTRA)

    pltpu.sync_copy(zrows_hbm, tmp_rows)
    pltpu.sync_copy(tmp_rows, s_sh.at[pl.ds(s * _RPT, _RPT)])
    if with_deg:
        pltpu.sync_copy(zdeg_hbm, tmp_deg)
        pltpu.sync_copy(tmp_deg, deg_sh.at[pl.ds(s * _RPT, _RPT)])
        pltpu.sync_copy(ones_hbm, ones_v)

    pltpu.sync_copy(src_hbm.at[pl.ds(base, _BASECH)],
                    src_v.at[pl.ds(0, _BASECH)])
    pltpu.sync_copy(dst_hbm.at[pl.ds(base, _BASECH)],
                    dst_v.at[pl.ds(0, _BASECH)])

    @pl.when(wid < _EXTRA)
    def _stage_extra():
        pltpu.sync_copy(src_hbm.at[pl.ds(base + _BASECH, 1)],
                        src_v.at[pl.ds(_BASECH, 1)])
        pltpu.sync_copy(dst_hbm.at[pl.ds(base + _BASECH, 1)],
                        dst_v.at[pl.ds(_BASECH, 1)])

    plsc.subcore_barrier()

    for p in range(_NB - 1):
        pltpu.async_copy(xl_hbm.at[src_v.at[p]], rows_v.at[p], gsem.at[p])

    def body(j, carry):
        b = lax.rem(j, _NB)
        ahead = j + (_NB - 1)
        fb = lax.rem(ahead, _NB)

        @pl.when(ahead < nch)
        def _fire_ahead():
            @pl.when(j >= 1)
            def _drain_scatter():
                pltpu.make_async_copy(
                    rows_v.at[fb], s_sh.at[dst_v.at[j]], ssem.at[fb]).wait()
            pltpu.async_copy(xl_hbm.at[src_v.at[ahead]], rows_v.at[fb],
                             gsem.at[fb])

        @pl.when(j < nch)
        def _consume():
            pltpu.make_async_copy(xl_hbm.at[src_v.at[j]], rows_v.at[b],
                                  gsem.at[b]).wait()
            pltpu.async_copy(rows_v.at[b], s_sh.at[dst_v.at[j]], ssem.at[b],
                             add=True)
            if with_deg:
                @pl.when(j >= _DLAG)
                def _drain_deg():
                    pltpu.make_async_copy(ones_v, deg_sh.at[dst_v.at[j]],
                                          dsem).wait()
                pltpu.async_copy(ones_v, deg_sh.at[dst_v.at[j]], dsem,
                                 add=True)
        return carry

    lax.fori_loop(0, _MAXCH, body, 0)
    for b in range(_NB):
        pltpu.make_async_copy(rows_v.at[b], s_sh.at[dst_v.at[0]],
                              ssem.at[b]).wait()
    if with_deg:
        for _ in range(_DLAG):
            pltpu.make_async_copy(ones_v, deg_sh.at[dst_v.at[0]],
                                  dsem).wait()
    plsc.subcore_barrier()

    pltpu.sync_copy(s_sh.at[pl.ds(s * _RPT, _RPT)], tmp_rows)
    pltpu.sync_copy(tmp_rows, s_out.at[c, pl.ds(s * _RPT, _RPT)])
    if with_deg:
        pltpu.sync_copy(deg_sh.at[pl.ds(s * _RPT, _RPT)], tmp_deg)
        pltpu.sync_copy(tmp_deg, deg_out.at[c, pl.ds(s * _RPT, _RPT)])


def _make_seg_kernel(with_deg):
    mesh = plsc.VectorSubcoreMesh(core_axis_name="c", subcore_axis_name="s")
    if with_deg:
        out_type = (jax.ShapeDtypeStruct((_NC, _NPAD, _H), jnp.float32),
                    jax.ShapeDtypeStruct((_NC, _NPAD), jnp.float32))
    else:
        out_type = jax.ShapeDtypeStruct((_NC, _NPAD, _H), jnp.float32)
    scratch = [pltpu.VMEM_SHARED((_NPAD, _H), jnp.float32)]
    if with_deg:
        scratch.append(pltpu.VMEM_SHARED((_NPAD,), jnp.float32))
    scratch += [
        pltpu.VMEM((_MAXCH, _CHUNK), jnp.int32),
        pltpu.VMEM((_MAXCH, _CHUNK), jnp.int32),
        pltpu.VMEM((_NB, _CHUNK, _H), jnp.float32),
    ]
    if with_deg:
        scratch.append(pltpu.VMEM((_CHUNK,), jnp.float32))
    scratch.append(pltpu.VMEM((_RPT, _H), jnp.float32))
    if with_deg:
        scratch.append(pltpu.VMEM((_RPT,), jnp.float32))
    scratch += [pltpu.SemaphoreType.DMA((_NB,)),
                pltpu.SemaphoreType.DMA((_NB,)),
                pltpu.SemaphoreType.DMA]
    return pl.kernel(functools.partial(_seg_body, with_deg),
                     out_type=out_type, mesh=mesh, scratch_types=scratch,
                     compiler_params=pltpu.CompilerParams(
                         use_tc_tiling_on_sc=False))


_seg_with_deg = _make_seg_kernel(True)
_seg_no_deg = _make_seg_kernel(False)


def _proj_body(x_ref, wl_ref, wr_ref, xl_ref, xr_ref):
    x = x_ref[...]
    xl_ref[...] = jnp.dot(x, wl_ref[...], preferred_element_type=jnp.float32)
    xr_ref[...] = jnp.dot(x, wr_ref[...], preferred_element_type=jnp.float32)


def _layer1_body(s1_ref, deg_ref, xr_ref, b_ref, h_ref, dinv_ref):
    deg = deg_ref[0] + deg_ref[1]
    dinv = 1.0 / jnp.maximum(deg, 1.0)
    s1 = s1_ref[0] + s1_ref[1]
    h_ref[...] = jnp.maximum(s1 * dinv + b_ref[...] + xr_ref[...], 0.0)
    dinv_ref[...] = dinv


def _layer2_body(s2_ref, dinv_ref, h_ref, wcat_ref, b_ref, o_ref):
    agg = (s2_ref[0, :_N] + s2_ref[1, :_N]) * dinv_ref[...]
    cat_t = jnp.concatenate([agg, h_ref[...]], axis=1).T
    z = (jnp.dot(wcat_ref[...], cat_t, preferred_element_type=jnp.float32)
         + b_ref[...])
    m = jnp.max(z, axis=0, keepdims=True)
    lse = jnp.log(jnp.sum(jnp.exp(z - m), axis=0, keepdims=True)) + m
    o_ref[...] = z - lse


_BN = 2000


def kernel(x, edge_index, W_l1, b_l1, W_r1, W_l2, b_l2, W_r2):
    f32 = jnp.float32
    src2d = edge_index[0].reshape(_TOTCH, _CHUNK)
    dst2d = edge_index[1].reshape(_TOTCH, _CHUNK)

    xl, xr = pl.pallas_call(
        _proj_body,
        grid=(_N // _BN,),
        in_specs=[pl.BlockSpec((_BN, _D), lambda i: (i, 0)),
                  pl.BlockSpec((_D, _H), lambda i: (0, 0)),
                  pl.BlockSpec((_D, _H), lambda i: (0, 0))],
        out_specs=[pl.BlockSpec((_BN, _H), lambda i: (i, 0)),
                   pl.BlockSpec((_BN, _H), lambda i: (i, 0))],
        out_shape=[jax.ShapeDtypeStruct((_N, _H), f32),
                   jax.ShapeDtypeStruct((_N, _H), f32)],
    )(x, W_l1, W_r1)

    ones_c = jnp.ones((_CHUNK,), f32)
    zdeg_c = jnp.zeros((_RPT,), f32)
    zrows_c = jnp.zeros((_RPT, _H), f32)

    s1p, degp = _seg_with_deg(xl, src2d, dst2d, ones_c, zdeg_c, zrows_c)
    degp = degp[:, :_N, None]

    h, dinv = pl.pallas_call(
        _layer1_body,
        grid=(_N // _BN,),
        in_specs=[pl.BlockSpec((2, _BN, _H), lambda i: (0, i, 0)),
                  pl.BlockSpec((2, _BN, 1), lambda i: (0, i, 0)),
                  pl.BlockSpec((_BN, _H), lambda i: (i, 0)),
                  pl.BlockSpec((1, _H), lambda i: (0, 0))],
        out_specs=[pl.BlockSpec((_BN, _H), lambda i: (i, 0)),
                   pl.BlockSpec((_BN, 1), lambda i: (i, 0))],
        out_shape=[jax.ShapeDtypeStruct((_N, _H), f32),
                   jax.ShapeDtypeStruct((_N, 1), f32)],
    )(s1p, degp, xr, b_l1.reshape(1, _H))

    s2p = _seg_no_deg(h, src2d, dst2d, ones_c, zdeg_c, zrows_c)

    w_cat_t = jnp.concatenate([W_l2, W_r2], axis=0).T
    out_t = pl.pallas_call(
        _layer2_body,
        out_shape=jax.ShapeDtypeStruct((_C, _N), f32),
    )(s2p, dinv, h, w_cat_t, b_l2.reshape(_C, 1))
    return out_t.T
